# baseline (speedup 1.0000x reference)
import functools

import jax
import jax.numpy as jnp
from jax import lax
from jax.experimental import pallas as pl
from jax.experimental.pallas import tpu as pltpu
from jax.experimental.pallas import tpu_sc as plsc

B = 8
C = 128
N = 2048
K = 20
EPS = 1e-5
SLOPE = 0.2

R = 256
NT = N // R
P = B * N


def _leaky(y):
    return jnp.where(y >= 0, y, SLOPE * y)


def _stage_a_body(xt_ref, xx_ref, idx_ref):
    b = pl.program_id(0)
    t = pl.program_id(1)
    xt = xt_ref[0]
    xr = xt_ref[0, pl.ds(t * R, R), :]

    g2 = lax.dot_general(xr, xt, (((1,), (1,)), ((), ())),
                         preferred_element_type=jnp.float32)
    xx = xx_ref[0, 0, :]
    xx_r = xx_ref[0, 0, pl.ds(t * R, R)]
    inner = -2.0 * g2
    d = (0.0 - xx_r[:, None]) - inner - xx[None, :]

    ii = lax.broadcasted_iota(jnp.int32, (R, N), 1)
    cols = []
    for _ in range(K):
        m = jnp.max(d, axis=1, keepdims=True)
        am = jnp.min(jnp.where(d == m, ii, N), axis=1, keepdims=True)
        cols.append(am)
        d = jnp.where(ii == am, -jnp.inf, d)
    idx_ref[0] = jnp.concatenate(cols, axis=1) + b * N


def _stage_a(xt, xx):
    return pl.pallas_call(
        _stage_a_body,
        grid=(B, NT),
        in_specs=[
            pl.BlockSpec((1, N, C), lambda b, t: (b, 0, 0)),
            pl.BlockSpec((1, 1, N), lambda b, t: (b, 0, 0)),
        ],
        out_specs=pl.BlockSpec((1, R, K), lambda b, t: (b, t, 0)),
        out_shape=jax.ShapeDtypeStruct((B, N, K), jnp.int32),
    )(xt, xx)


NW = 32
PPW = P // NW
G = 16
NCH = PPW // G
GK = G * K


def _sc_gather_body(x_hbm, idx_hbm, g_hbm, idx_v0, idx_v1, rows_v0, rows_v1,
                    sg0, sg1, so0, so1):
    info = plsc.get_sparse_core_info()
    nc = info.num_cores
    wid = lax.axis_index("s") * nc + lax.axis_index("c")

    def pair_body(p, carry):
        base0 = wid * PPW + (2 * p) * G
        base1 = base0 + G
        pltpu.sync_copy(idx_hbm.at[pl.ds(base0 * K, GK)], idx_v0)
        g0 = pltpu.async_copy(x_hbm.at[idx_v0], rows_v0, sg0)
        pltpu.sync_copy(idx_hbm.at[pl.ds(base1 * K, GK)], idx_v1)
        g1 = pltpu.async_copy(x_hbm.at[idx_v1], rows_v1, sg1)
        g0.wait()
        o0 = pltpu.async_copy(rows_v0, g_hbm.at[pl.ds(base0 * K, GK)], so0)
        g1.wait()
        o1 = pltpu.async_copy(rows_v1, g_hbm.at[pl.ds(base1 * K, GK)], so1)
        o0.wait()
        o1.wait()
        return carry

    lax.fori_loop(0, NCH // 2, pair_body, 0)


def _gather_rows(x_flat, idx_flat):
    mesh = plsc.VectorSubcoreMesh(core_axis_name="c", subcore_axis_name="s")
    f = functools.partial(
        pl.kernel,
        mesh=mesh,
        out_type=jax.ShapeDtypeStruct((P * K, C), jnp.float32),
        scratch_types=[
            pltpu.VMEM((GK,), jnp.int32),
            pltpu.VMEM((GK,), jnp.int32),
            pltpu.VMEM((GK, C), jnp.float32),
            pltpu.VMEM((GK, C), jnp.float32),
            pltpu.SemaphoreType.DMA,
            pltpu.SemaphoreType.DMA,
            pltpu.SemaphoreType.DMA,
            pltpu.SemaphoreType.DMA,
        ],
    )(_sc_gather_body)
    return f(x_flat, idx_flat)


R2 = 128
NT2 = N // R2


def _edge_body(g_ref, xt_ref, wt_ref, m_ref, sums_ref):
    b = pl.program_id(0)
    t = pl.program_id(1)
    xj = g_ref[0]
    xi = xt_ref[0]
    xi_rep = jnp.broadcast_to(xi[:, None, :], (R2, K, C)).reshape(R2 * K, C)
    feat = jnp.concatenate([xj - xi_rep, xi_rep], axis=1)
    y = jnp.dot(feat, wt_ref[...], preferred_element_type=jnp.float32)
    y3 = y.reshape(R2, K, C)
    m_ref[0] = jnp.max(y3, axis=1)
    s1 = jnp.sum(y, axis=0)
    s2 = jnp.sum(y * y, axis=0)
    new = jnp.stack([s1, s2])
    first = jnp.logical_and(b == 0, t == 0)

    @pl.when(first)
    def _():
        sums_ref[...] = new

    @pl.when(jnp.logical_not(first))
    def _():
        sums_ref[...] = sums_ref[...] + new


def _edge_matmul(gathered, xt, w_t):
    return pl.pallas_call(
        _edge_body,
        grid=(B, NT2),
        in_specs=[
            pl.BlockSpec((1, R2 * K, C), lambda b, t: (b, t, 0)),
            pl.BlockSpec((1, R2, C), lambda b, t: (b, t, 0)),
            pl.BlockSpec((2 * C, C), lambda b, t: (0, 0)),
        ],
        out_specs=[
            pl.BlockSpec((1, R2, C), lambda b, t: (b, t, 0)),
            pl.BlockSpec((2, C), lambda b, t: (0, 0)),
        ],
        out_shape=[
            jax.ShapeDtypeStruct((B, N, C), jnp.float32),
            jax.ShapeDtypeStruct((2, C), jnp.float32),
        ],
    )(gathered, xt, w_t)


def _stage_b_body(m_ref, part_ref, g_ref, b_ref, out_ref):
    s1 = part_ref[0, :]
    s2 = part_ref[1, :]
    cnt = float(B * N * K)
    mean = s1 / cnt
    var = s2 / cnt - mean * mean
    inv = g_ref[...] * lax.rsqrt(var + EPS)
    xn = (m_ref[0] - mean[None, :]) * inv[None, :] + b_ref[...][None, :]
    out_ref[0] = _leaky(xn)


def _stage_b(m, parts, g, bb):
    return pl.pallas_call(
        _stage_b_body,
        grid=(B,),
        in_specs=[
            pl.BlockSpec((1, N, C), lambda b: (b, 0, 0)),
            pl.BlockSpec((2, C), lambda b: (0, 0)),
            pl.BlockSpec((C,), lambda b: (0,)),
            pl.BlockSpec((C,), lambda b: (0,)),
        ],
        out_specs=pl.BlockSpec((1, N, C), lambda b: (b, 0, 0)),
        out_shape=jax.ShapeDtypeStruct((B, N, C), jnp.float32),
    )(m, parts, g, bb)


def _final_body(x1_ref, x2_ref, x3_ref, wp_ref, mx_ref, sums_ref):
    b = pl.program_id(0)
    t = pl.program_id(1)
    cat = jnp.concatenate([x1_ref[0], x2_ref[0], x3_ref[0]], axis=1)
    yp = jnp.dot(cat, wp_ref[...], preferred_element_type=jnp.float32,
                 precision=lax.Precision.HIGHEST)
    rmax = jnp.max(yp, axis=0)
    rsum = jnp.sum(yp, axis=0)
    rsq = jnp.sum(yp * yp, axis=0)
    new = jnp.stack([rsum, rsq])

    @pl.when(t == 0)
    def _():
        mx_ref[...] = rmax[None, None, :]

    @pl.when(t != 0)
    def _():
        mx_ref[...] = jnp.maximum(mx_ref[...], rmax[None, None, :])

    first = jnp.logical_and(b == 0, t == 0)

    @pl.when(first)
    def _():
        sums_ref[...] = new

    @pl.when(jnp.logical_not(first))
    def _():
        sums_ref[...] = sums_ref[...] + new


def _final(x1, x2, x3, wp_t):
    return pl.pallas_call(
        _final_body,
        grid=(B, NT),
        in_specs=[
            pl.BlockSpec((1, R, C), lambda b, t: (b, t, 0)),
            pl.BlockSpec((1, R, C), lambda b, t: (b, t, 0)),
            pl.BlockSpec((1, R, C), lambda b, t: (b, t, 0)),
            pl.BlockSpec((3 * C, C), lambda b, t: (0, 0)),
        ],
        out_specs=[
            pl.BlockSpec((1, 1, C), lambda b, t: (b, 0, 0)),
            pl.BlockSpec((2, C), lambda b, t: (0, 0)),
        ],
        out_shape=[
            jax.ShapeDtypeStruct((B, 1, C), jnp.float32),
            jax.ShapeDtypeStruct((2, C), jnp.float32),
        ],
    )(x1, x2, x3, wp_t)


def _epilogue_body(mx_ref, sums_ref, g_ref, b_ref, out_ref):
    cnt = float(B * N)
    mean = sums_ref[0, :] / cnt
    var = sums_ref[1, :] / cnt - mean * mean
    inv = g_ref[...] * lax.rsqrt(var + EPS)
    y = (mx_ref[...] - mean[None, None, :]) * inv[None, None, :]
    out_ref[...] = _leaky(y + b_ref[...][None, None, :])


def _epilogue(mx, sums, gp, bp):
    return pl.pallas_call(
        _epilogue_body,
        out_shape=jax.ShapeDtypeStruct((B, 1, C), jnp.float32),
    )(mx, sums, gp, bp)


def _edge_stage(xt, xx, W, g, bb):
    idx = _stage_a(xt, xx)
    gathered = _gather_rows(xt.reshape(P, C), idx.reshape(P * K))
    m, sums = _edge_matmul(gathered.reshape(B, N * K, C), xt,
                           jnp.transpose(W))
    return _stage_b(m, sums, g, bb)


def kernel(x, pos, W1, g1, b1, W2, g2, b2, W3, g3, b3, Wp, gp, bp):
    xt = jnp.transpose(x, (0, 2, 1))
    xx0 = jnp.sum(x * x, axis=1)[:, None, :]
    x1 = _edge_stage(xt, xx0, W1, g1, b1)
    xx1 = jnp.sum(x1 * x1, axis=2)[:, None, :]
    x2 = _edge_stage(x1, xx1, W2, g2, b2)
    xx2 = jnp.sum(x2 * x2, axis=2)[:, None, :]
    x3 = _edge_stage(x2, xx2, W3, g3, b3)
    mx, sums = _final(x1, x2, x3, jnp.transpose(Wp))
    return _epilogue(mx, sums, gp, bp)


# speedup vs baseline: 1.0727x; 1.0727x over previous
import functools

import jax
import jax.numpy as jnp
from jax import lax
from jax.experimental import pallas as pl
from jax.experimental.pallas import tpu as pltpu
from jax.experimental.pallas import tpu_sc as plsc

B = 8
C = 128
N = 2048
K = 20
EPS = 1e-5
SLOPE = 0.2

R = 256
NT = N // R
P = B * N


def _leaky(y):
    return jnp.where(y >= 0, y, SLOPE * y)


def _stage_a_body(xt_ref, xx_ref, idx_ref):
    b = pl.program_id(0)
    t = pl.program_id(1)
    xt = xt_ref[0]
    xr = xt_ref[0, pl.ds(t * R, R), :]

    g2 = lax.dot_general(xr, xt, (((1,), (1,)), ((), ())),
                         preferred_element_type=jnp.float32)
    xx = xx_ref[0, 0, :]
    xx_r = xx_ref[0, 0, pl.ds(t * R, R)]
    inner = -2.0 * g2
    d = (0.0 - xx_r[:, None]) - inner - xx[None, :]

    ii = lax.broadcasted_iota(jnp.int32, (R, N), 1)
    cols = []
    for _ in range(K):
        m = jnp.max(d, axis=1, keepdims=True)
        am = jnp.min(jnp.where(d == m, ii, N), axis=1, keepdims=True)
        cols.append(am)
        d = jnp.where(ii == am, -jnp.inf, d)
    idx_ref[0] = jnp.concatenate(cols, axis=1) + b * N


def _stage_a(xt, xx):
    nb = xt.shape[0]
    return pl.pallas_call(
        _stage_a_body,
        grid=(nb, NT),
        in_specs=[
            pl.BlockSpec((1, N, C), lambda b, t: (b, 0, 0)),
            pl.BlockSpec((1, 1, N), lambda b, t: (b, 0, 0)),
        ],
        out_specs=pl.BlockSpec((1, R, K), lambda b, t: (b, t, 0)),
        out_shape=jax.ShapeDtypeStruct((nb, N, K), jnp.int32),
    )(xt, xx)


NW = 32
PPW = P // NW
G = 16
NCH = PPW // G
GK = G * K


def _sc_gather_body(ppw, x_hbm, idx_hbm, g_hbm, idx_v0, idx_v1, rows_v0,
                    rows_v1, sg0, sg1, so0, so1):
    info = plsc.get_sparse_core_info()
    nc = info.num_cores
    wid = lax.axis_index("s") * nc + lax.axis_index("c")

    def pair_body(p, carry):
        base0 = wid * ppw + (2 * p) * G
        base1 = base0 + G
        pltpu.sync_copy(idx_hbm.at[pl.ds(base0 * K, GK)], idx_v0)
        g0 = pltpu.async_copy(x_hbm.at[idx_v0], rows_v0, sg0)
        pltpu.sync_copy(idx_hbm.at[pl.ds(base1 * K, GK)], idx_v1)
        g1 = pltpu.async_copy(x_hbm.at[idx_v1], rows_v1, sg1)
        g0.wait()
        o0 = pltpu.async_copy(rows_v0, g_hbm.at[pl.ds(base0 * K, GK)], so0)
        g1.wait()
        o1 = pltpu.async_copy(rows_v1, g_hbm.at[pl.ds(base1 * K, GK)], so1)
        o0.wait()
        o1.wait()
        return carry

    lax.fori_loop(0, ppw // (2 * G), pair_body, 0)


def _gather_rows(x_flat, idx_flat):
    npts = x_flat.shape[0]
    mesh = plsc.VectorSubcoreMesh(core_axis_name="c", subcore_axis_name="s")
    f = functools.partial(
        pl.kernel,
        mesh=mesh,
        out_type=jax.ShapeDtypeStruct((npts * K, C), jnp.float32),
        scratch_types=[
            pltpu.VMEM((GK,), jnp.int32),
            pltpu.VMEM((GK,), jnp.int32),
            pltpu.VMEM((GK, C), jnp.float32),
            pltpu.VMEM((GK, C), jnp.float32),
            pltpu.SemaphoreType.DMA,
            pltpu.SemaphoreType.DMA,
            pltpu.SemaphoreType.DMA,
            pltpu.SemaphoreType.DMA,
        ],
    )(functools.partial(_sc_gather_body, npts // NW))
    return f(x_flat, idx_flat)


R2 = 128
NT2 = N // R2


def _edge_body(g_ref, xt_ref, wt_ref, m_ref, sums_ref):
    b = pl.program_id(0)
    t = pl.program_id(1)
    xj = g_ref[0]
    xi = xt_ref[0]
    xi_rep = jnp.broadcast_to(xi[:, None, :], (R2, K, C)).reshape(R2 * K, C)
    feat = jnp.concatenate([xj - xi_rep, xi_rep], axis=1)
    y = jnp.dot(feat, wt_ref[...], preferred_element_type=jnp.float32)
    y3 = y.reshape(R2, K, C)
    m_ref[0] = jnp.max(y3, axis=1)
    s1 = jnp.sum(y, axis=0)
    s2 = jnp.sum(y * y, axis=0)
    new = jnp.stack([s1, s2])
    first = jnp.logical_and(b == 0, t == 0)

    @pl.when(first)
    def _():
        sums_ref[...] = new

    @pl.when(jnp.logical_not(first))
    def _():
        sums_ref[...] = sums_ref[...] + new


def _edge_matmul(gathered, xt, w_t):
    nb = xt.shape[0]
    return pl.pallas_call(
        _edge_body,
        grid=(nb, NT2),
        in_specs=[
            pl.BlockSpec((1, R2 * K, C), lambda b, t: (b, t, 0)),
            pl.BlockSpec((1, R2, C), lambda b, t: (b, t, 0)),
            pl.BlockSpec((2 * C, C), lambda b, t: (0, 0)),
        ],
        out_specs=[
            pl.BlockSpec((1, R2, C), lambda b, t: (b, t, 0)),
            pl.BlockSpec((2, C), lambda b, t: (0, 0)),
        ],
        out_shape=[
            jax.ShapeDtypeStruct((nb, N, C), jnp.float32),
            jax.ShapeDtypeStruct((2, C), jnp.float32),
        ],
    )(gathered, xt, w_t)


def _stage_b_body(m_ref, part_ref, g_ref, b_ref, out_ref):
    s1 = part_ref[0, :]
    s2 = part_ref[1, :]
    cnt = float(B * N * K)
    mean = s1 / cnt
    var = s2 / cnt - mean * mean
    inv = g_ref[...] * lax.rsqrt(var + EPS)
    xn = (m_ref[0] - mean[None, :]) * inv[None, :] + b_ref[...][None, :]
    out_ref[0] = _leaky(xn)


def _stage_b(m, parts, g, bb):
    return pl.pallas_call(
        _stage_b_body,
        grid=(B,),
        in_specs=[
            pl.BlockSpec((1, N, C), lambda b: (b, 0, 0)),
            pl.BlockSpec((2, C), lambda b: (0, 0)),
            pl.BlockSpec((C,), lambda b: (0,)),
            pl.BlockSpec((C,), lambda b: (0,)),
        ],
        out_specs=pl.BlockSpec((1, N, C), lambda b: (b, 0, 0)),
        out_shape=jax.ShapeDtypeStruct((B, N, C), jnp.float32),
    )(m, parts, g, bb)


def _final_body(x1_ref, x2_ref, x3_ref, wp_ref, mx_ref, sums_ref):
    b = pl.program_id(0)
    t = pl.program_id(1)
    cat = jnp.concatenate([x1_ref[0], x2_ref[0], x3_ref[0]], axis=1)
    yp = jnp.dot(cat, wp_ref[...], preferred_element_type=jnp.float32,
                 precision=lax.Precision.HIGHEST)
    rmax = jnp.max(yp, axis=0)
    rsum = jnp.sum(yp, axis=0)
    rsq = jnp.sum(yp * yp, axis=0)
    new = jnp.stack([rsum, rsq])

    @pl.when(t == 0)
    def _():
        mx_ref[...] = rmax[None, None, :]

    @pl.when(t != 0)
    def _():
        mx_ref[...] = jnp.maximum(mx_ref[...], rmax[None, None, :])

    first = jnp.logical_and(b == 0, t == 0)

    @pl.when(first)
    def _():
        sums_ref[...] = new

    @pl.when(jnp.logical_not(first))
    def _():
        sums_ref[...] = sums_ref[...] + new


def _final(x1, x2, x3, wp_t):
    return pl.pallas_call(
        _final_body,
        grid=(B, NT),
        in_specs=[
            pl.BlockSpec((1, R, C), lambda b, t: (b, t, 0)),
            pl.BlockSpec((1, R, C), lambda b, t: (b, t, 0)),
            pl.BlockSpec((1, R, C), lambda b, t: (b, t, 0)),
            pl.BlockSpec((3 * C, C), lambda b, t: (0, 0)),
        ],
        out_specs=[
            pl.BlockSpec((1, 1, C), lambda b, t: (b, 0, 0)),
            pl.BlockSpec((2, C), lambda b, t: (0, 0)),
        ],
        out_shape=[
            jax.ShapeDtypeStruct((B, 1, C), jnp.float32),
            jax.ShapeDtypeStruct((2, C), jnp.float32),
        ],
    )(x1, x2, x3, wp_t)


def _epilogue_body(mx_ref, sums_ref, g_ref, b_ref, out_ref):
    cnt = float(B * N)
    mean = sums_ref[0, :] / cnt
    var = sums_ref[1, :] / cnt - mean * mean
    inv = g_ref[...] * lax.rsqrt(var + EPS)
    y = (mx_ref[...] - mean[None, None, :]) * inv[None, None, :]
    out_ref[...] = _leaky(y + b_ref[...][None, None, :])


def _epilogue(mx, sums, gp, bp):
    return pl.pallas_call(
        _epilogue_body,
        out_shape=jax.ShapeDtypeStruct((B, 1, C), jnp.float32),
    )(mx, sums, gp, bp)


def _edge_stage(xt, xx, W, g, bb):
    hb = B // 2
    w_t = jnp.transpose(W)
    ms, sums = [], []
    for h in range(2):
        xth = xt[h * hb:(h + 1) * hb]
        xxh = xx[h * hb:(h + 1) * hb]
        idx = _stage_a(xth, xxh)
        gathered = _gather_rows(xth.reshape(hb * N, C),
                                idx.reshape(hb * N * K))
        m, s = _edge_matmul(gathered.reshape(hb, N * K, C), xth, w_t)
        ms.append(m)
        sums.append(s)
    return _stage_b(jnp.concatenate(ms, axis=0), sums[0] + sums[1], g, bb)


def kernel(x, pos, W1, g1, b1, W2, g2, b2, W3, g3, b3, Wp, gp, bp):
    xt = jnp.transpose(x, (0, 2, 1))
    xx0 = jnp.sum(x * x, axis=1)[:, None, :]
    x1 = _edge_stage(xt, xx0, W1, g1, b1)
    xx1 = jnp.sum(x1 * x1, axis=2)[:, None, :]
    x2 = _edge_stage(x1, xx1, W2, g2, b2)
    xx2 = jnp.sum(x2 * x2, axis=2)[:, None, :]
    x3 = _edge_stage(x2, xx2, W3, g3, b3)
    mx, sums = _final(x1, x2, x3, jnp.transpose(Wp))
    return _epilogue(mx, sums, gp, bp)
